# baseline (speedup 1.0000x reference)
import functools

import jax
import jax.numpy as jnp
from jax import lax
from jax.experimental import pallas as pl
from jax.experimental.pallas import tpu as pltpu
from jax.experimental.pallas import tpu_sc as plsc

BATCH = 16384
EMBED_DIM = 64

_info = plsc.get_sparse_core_info()
_NC, _NS = _info.num_cores, _info.num_subcores
_NW = _NC * _NS
_BPW = BATCH // _NW
_CH = 128
_NCHUNK = _BPW // _CH
_LANES = 16


def _make_kernel():
    mesh = plsc.VectorSubcoreMesh(core_axis_name="c", subcore_axis_name="s")

    @functools.partial(
        pl.kernel,
        mesh=mesh,
        out_type=(
            jax.ShapeDtypeStruct((BATCH, EMBED_DIM), jnp.float32),
            jax.ShapeDtypeStruct((BATCH, EMBED_DIM), jnp.float32),
        ),
        scratch_types=[
            pltpu.VMEM((_BPW,), jnp.int32),
            pltpu.VMEM((_BPW,), jnp.int32),
            pltpu.VMEM((2 * _CH, EMBED_DIM), jnp.float32),
            pltpu.SemaphoreType.DMA,
            pltpu.SemaphoreType.DMA,
        ],
    )
    def emb_kernel(e1_hbm, rel_hbm, tab_e_hbm, tab_r_hbm, out_e_hbm,
                   out_r_hbm, idx_e, idx_r, stage, sem, sem_w):
        wid = lax.axis_index("s") * _NC + lax.axis_index("c")
        base = wid * _BPW
        pltpu.sync_copy(e1_hbm.at[pl.ds(base, _BPW)], idx_e)
        pltpu.sync_copy(rel_hbm.at[pl.ds(base, _BPW)], idx_r)

        def wait_writeout(out_hbm):
            pltpu.make_async_copy(
                stage.at[pl.ds(0, _CH)],
                out_hbm.at[pl.ds(base, _CH)], sem_w).wait()

        def lookup_table(tab_hbm, idx, out_hbm):
            def chunk_body(k, carry):
                j0 = k * _CH
                o = (k % 2) * _CH

                @pl.when(k >= 2)
                def _free():
                    wait_writeout(out_hbm)

                for g in range(_CH // _LANES):
                    v = idx[pl.ds(j0 + g * _LANES, _LANES)]
                    v_blk = v >> 3
                    v_sub = v & 7
                    for lane in range(_LANES):
                        j = g * _LANES + lane
                        pltpu.make_async_copy(
                            tab_hbm.at[v_blk[lane], pl.ds(v_sub[lane], 1)],
                            stage.at[pl.ds(o + j, 1)], sem).start()
                pltpu.make_async_copy(
                    out_hbm.at[pl.ds(base, _CH)],
                    stage.at[pl.ds(0, _CH)], sem).wait()
                pltpu.make_async_copy(
                    stage.at[pl.ds(o, _CH)],
                    out_hbm.at[pl.ds(base + j0, _CH)], sem_w).start()
                return carry

            lax.fori_loop(0, _NCHUNK, chunk_body, None, unroll=False)
            for _ in range(min(2, _NCHUNK)):
                wait_writeout(out_hbm)

        lookup_table(tab_e_hbm, idx_e, out_e_hbm)
        lookup_table(tab_r_hbm, idx_r, out_r_hbm)

    return emb_kernel


_emb_kernel = _make_kernel()


def kernel(e1, rel, emb_e_weight, emb_rel_weight):
    e1_flat = e1.reshape(BATCH)
    rel_flat = rel.reshape(BATCH)
    tab_e = emb_e_weight.reshape(-1, 8, EMBED_DIM)
    tab_r = emb_rel_weight.reshape(-1, 8, EMBED_DIM)
    return _emb_kernel(e1_flat, rel_flat, tab_e, tab_r)
